# baseline (speedup 1.0000x reference)
import numpy as np
import jax
import jax.numpy as jnp
from jax import lax
from jax.experimental import pallas as pl
from jax.experimental.pallas import tpu as pltpu
from jax.experimental.pallas import tpu_sc as plsc

_N = 65536
_NW = 32
_PW = _N // _NW
_C = 256
_G = _C // 16
_NCHUNK = _PW // _C

_LX, _LOG2TX = 16, 19
_LR, _LOG2TR = 32, 21
_NMIN = 16
_BX = float(np.exp(np.log(2048.0 / _NMIN) / (_LX - 1)))
_BR = float(np.exp(np.log(2048.0 / _NMIN) / (_LR - 1)))
_RES_X = np.tile(np.array([float(int(np.floor(_NMIN * _BX ** l)))
                           for l in range(_LX)], np.float32)[:, None], (1, 16))
_RES_R = np.tile(np.array([float(int(np.floor(_NMIN * _BR ** l)))
                           for l in range(_LR)], np.float32)[:, None], (1, 16))
_P2 = 2654435761 - (1 << 32)
_P3 = 805459861

_CORNERS = [(cx, cy, cz) for cx in (0, 1) for cy in (0, 1) for cz in (0, 1)]


def _make_grid_encoder(n_levels, log2t, want_grad):
    t_sz = 1 << log2t
    mask = t_sz - 1
    no = 2 * n_levels

    outs = [jax.ShapeDtypeStruct((no, _N), jnp.float32)]
    if want_grad:
        outs.append(jax.ShapeDtypeStruct((3 * no, _N), jnp.float32))

    scratch = (
        [pltpu.VMEM((_C,), jnp.float32) for _ in range(3)]
        + [pltpu.VMEM((_C,), jnp.int32) for _ in range(16)]
        + [pltpu.VMEM((_C,), jnp.float32) for _ in range(32)]
        + [pltpu.VMEM((no, _C), jnp.float32)]
        + ([pltpu.VMEM((3 * no, _C), jnp.float32)] if want_grad else [])
        + [pltpu.VMEM((n_levels, 16), jnp.float32),
           pltpu.SemaphoreType.DMA, pltpu.SemaphoreType.DMA]
    )
    mesh = plsc.VectorSubcoreMesh(core_axis_name="c", subcore_axis_name="s")

    def body(*refs):
        xx_h, xy_h, xz_h, tab0_h, tab1_h, res_h = refs[0:6]
        pos = 6
        feat_h = refs[pos]; pos += 1
        if want_grad:
            g3_h = refs[pos]; pos += 1
        xv = refs[pos:pos + 3]; pos += 3
        idxr = refs[pos:pos + 16]; pos += 16
        rows = refs[pos:pos + 32]; pos += 32
        featb = refs[pos]; pos += 1
        if want_grad:
            g3b = refs[pos]; pos += 1
        resv = refs[pos]; pos += 1
        sems = refs[pos:pos + 2]; pos += 2

        wid = lax.axis_index("s") * 2 + lax.axis_index("c")
        base = wid * _PW
        pltpu.sync_copy(res_h, resv)

        def posw(g, resvec):
            out = []
            for ref in xv:
                xn = (ref[pl.ds(g * 16, 16)] + 1.0) * 0.5
                p = xn * resvec
                pi = p.astype(jnp.int32)
                w = p - pi.astype(jnp.float32)
                out.append((pi, w))
            return out

        def chunk_body(k, carry):
            rowstart = base + k * _C
            pltpu.sync_copy(xx_h.at[pl.ds(rowstart, _C)], xv[0])
            pltpu.sync_copy(xy_h.at[pl.ds(rowstart, _C)], xv[1])
            pltpu.sync_copy(xz_h.at[pl.ds(rowstart, _C)], xv[2])

            def fire(l, st):
                resvec = resv[l]
                lofs = l * t_sz

                def genidx(g, c3):
                    (pix, _), (piy, _), (piz, _) = posw(g, resvec)
                    hx = (pix, pix + 1)
                    hy0 = piy * _P2
                    hy = (hy0, hy0 + _P2)
                    hz0 = piz * _P3
                    hz = (hz0, hz0 + _P3)
                    s = pl.ds(g * 16, 16)
                    for c, (cx, cy, cz) in enumerate(_CORNERS):
                        idxr[8 * st + c][s] = ((((hx[cx] ^ hy[cy]) ^ hz[cz])
                                                & mask) + lofs)
                    return c3

                lax.fori_loop(0, _G, genidx, 0)
                for c in range(8):
                    pltpu.async_copy(tab0_h.at[idxr[8 * st + c]],
                                     rows[16 * st + 2 * c], sems[st])
                    pltpu.async_copy(tab1_h.at[idxr[8 * st + c]],
                                     rows[16 * st + 2 * c + 1], sems[st])

            def drain(st):
                for c in range(8):
                    pltpu.make_async_copy(tab0_h.at[idxr[8 * st + c]],
                                          rows[16 * st + 2 * c],
                                          sems[st]).wait()
                    pltpu.make_async_copy(tab1_h.at[idxr[8 * st + c]],
                                          rows[16 * st + 2 * c + 1],
                                          sems[st]).wait()

            def accum_level(l, st):
                resvec = resv[l]

                def accum(g, c3):
                    (pix, wx1), (piy, wy1), (piz, wz1) = posw(g, resvec)
                    wx0, wy0, wz0 = 1.0 - wx1, 1.0 - wy1, 1.0 - wz1
                    wx = (wx0, wx1)
                    wyz = ((wy0 * wz0, wy0 * wz1), (wy1 * wz0, wy1 * wz1))
                    if want_grad:
                        wxz = ((wx0 * wz0, wx0 * wz1), (wx1 * wz0, wx1 * wz1))
                        wxy = ((wx0 * wy0, wx0 * wy1), (wx1 * wy0, wx1 * wy1))
                    s = pl.ds(g * 16, 16)
                    zf = wx1 * 0.0
                    acc0 = zf
                    acc1 = zf
                    if want_grad:
                        gx0 = zf; gx1 = zf
                        gy0 = zf; gy1 = zf
                        gz0 = zf; gz1 = zf
                    for c, (cx, cy, cz) in enumerate(_CORNERS):
                        f0 = rows[16 * st + 2 * c][s]
                        f1 = rows[16 * st + 2 * c + 1][s]
                        w3 = wx[cx] * wyz[cy][cz]
                        acc0 = acc0 + w3 * f0
                        acc1 = acc1 + w3 * f1
                        if want_grad:
                            ta = wyz[cy][cz]
                            tb = wxz[cx][cz]
                            tc = wxy[cx][cy]
                            if cx:
                                gx0 = gx0 + ta * f0
                                gx1 = gx1 + ta * f1
                            else:
                                gx0 = gx0 - ta * f0
                                gx1 = gx1 - ta * f1
                            if cy:
                                gy0 = gy0 + tb * f0
                                gy1 = gy1 + tb * f1
                            else:
                                gy0 = gy0 - tb * f0
                                gy1 = gy1 - tb * f1
                            if cz:
                                gz0 = gz0 + tc * f0
                                gz1 = gz1 + tc * f1
                            else:
                                gz0 = gz0 - tc * f0
                                gz1 = gz1 - tc * f1
                    featb[2 * l, s] = acc0
                    featb[2 * l + 1, s] = acc1
                    if want_grad:
                        fres = resvec * 0.5
                        g3b[2 * l, s] = gx0 * fres
                        g3b[2 * l + 1, s] = gx1 * fres
                        g3b[no + 2 * l, s] = gy0 * fres
                        g3b[no + 2 * l + 1, s] = gy1 * fres
                        g3b[2 * no + 2 * l, s] = gz0 * fres
                        g3b[2 * no + 2 * l + 1, s] = gz1 * fres
                    return c3

                lax.fori_loop(0, _G, accum, 0)

            fire(0, 0)

            def pair_body(i, carry2):
                l0 = 2 * i
                fire(l0 + 1, 1)
                drain(0)
                accum_level(l0, 0)

                @pl.when(i + 1 < n_levels // 2)
                def _():
                    fire(l0 + 2, 0)

                drain(1)
                accum_level(l0 + 1, 1)
                return carry2

            lax.fori_loop(0, n_levels // 2, pair_body, 0)
            pltpu.sync_copy(featb, feat_h.at[:, pl.ds(rowstart, _C)])
            if want_grad:
                pltpu.sync_copy(g3b, g3_h.at[:, pl.ds(rowstart, _C)])
            return carry

        lax.fori_loop(0, _NCHUNK, chunk_body, 0)

    return pl.kernel(body, out_type=tuple(outs), mesh=mesh,
                     scratch_types=scratch)


_BN = 2048
_NBLK = _N // _BN


def _softplus(v):
    return jnp.maximum(v, 0.0) + jnp.log(1.0 + jnp.exp(-jnp.abs(v)))


def _sigmoid(v):
    return 1.0 / (1.0 + jnp.exp(-v))


def _tc_body(featx, g3, featr, xb, db, ea, w1, b1, w2, w2r, w1t, w1x, w1d,
             w1f, w1a, rgbw2, nw1, nw2, sw1, sw2, b2,
             sig_o, rgb_o, nraw_o, npred_o, sem_o, cnt_o):
    f = featx[...].T
    pre1 = jnp.dot(f, w1[...], preferred_element_type=jnp.float32) + b1[...]
    h1 = _softplus(pre1)
    h2 = jnp.dot(h1, w2[...], preferred_element_type=jnp.float32) + b2[...]
    sig_o[...] = _softplus(100.0 * h2) / 100.0

    dh1 = _sigmoid(100.0 * h2) * w2r[...]
    dpre1 = dh1 * _sigmoid(pre1)
    g_f = jnp.dot(dpre1, w1t[...], preferred_element_type=jnp.float32)
    g3v = g3[...].T
    no = g_f.shape[1]
    gx = jnp.sum(g3v[:, 0:no] * g_f, axis=1, keepdims=True)
    gy = jnp.sum(g3v[:, no:2 * no] * g_f, axis=1, keepdims=True)
    gz = jnp.sum(g3v[:, 2 * no:3 * no] * g_f, axis=1, keepdims=True)
    gn = jnp.maximum(jnp.sqrt(gx * gx + gy * gy + gz * gz), 1e-6)
    nraw_o[:, 0:1] = -gx / gn
    nraw_o[:, 1:2] = -gy / gn
    nraw_o[:, 2:3] = -gz / gn
    cntv = jnp.sum(jnp.isinf(gx).astype(jnp.int32) +
                   jnp.isinf(gy).astype(jnp.int32) +
                   jnp.isinf(gz).astype(jnp.int32))
    cnt_o[...] = lax.broadcast_in_dim(cntv, (1, 1, 128), ())

    fr = featr[...].T
    hp = jnp.dot(jnp.maximum(jnp.dot(fr, nw1[...],
                                     preferred_element_type=jnp.float32), 0.0),
                 nw2[...], preferred_element_type=jnp.float32)
    hn = jnp.maximum(jnp.sqrt(jnp.sum(hp * hp, axis=1, keepdims=True)), 1e-6)
    npred_o[...] = -hp / hn

    sm = jnp.dot(jnp.maximum(jnp.dot(fr, sw1[...],
                                     preferred_element_type=jnp.float32), 0.0),
                 sw2[...], preferred_element_type=jnp.float32)
    sm = sm - jnp.max(sm, axis=1, keepdims=True)
    es = jnp.exp(sm)
    sem_o[...] = es / jnp.sum(es, axis=1, keepdims=True)

    dv = db[...]
    dn = dv / jnp.maximum(jnp.sqrt(jnp.sum(dv * dv, axis=1, keepdims=True)),
                          1e-6)
    u = (dn + 1.0) / 2.0
    sx = 2.0 * u[:, 0:1] - 1.0
    sy = 2.0 * u[:, 1:2] - 1.0
    sz = 2.0 * u[:, 2:3] - 1.0
    x2, y2, z2 = sx * sx, sy * sy, sz * sz
    xy, yz, xz = sx * sy, sy * sz, sx * sz
    denc = [
        None,
        -0.48860251190291987 * sy,
        0.48860251190291987 * sz,
        -0.48860251190291987 * sx,
        1.0925484305920792 * xy,
        -1.0925484305920792 * yz,
        0.94617469575755997 * z2 - 0.31539156525251999,
        -1.0925484305920792 * xz,
        0.54627421529603959 * (x2 - y2),
        0.59004358992664352 * sy * (-3.0 * x2 + y2),
        2.8906114426405538 * xy * sz,
        0.45704579946446572 * sy * (1.0 - 5.0 * z2),
        0.3731763325901154 * sz * (5.0 * z2 - 3.0),
        0.45704579946446572 * sx * (1.0 - 5.0 * z2),
        1.4453057213202769 * sz * (x2 - y2),
        0.59004358992664352 * sx * (x2 - 3.0 * y2),
    ]
    pre = (jnp.dot(xb[...], w1x[...], preferred_element_type=jnp.float32)
           + jnp.dot(fr, w1f[...], preferred_element_type=jnp.float32)
           + jnp.dot(ea[...], w1a[...], preferred_element_type=jnp.float32))
    pre = pre + 0.28209479177387814 * w1d[0:1, :]
    for k in range(1, 16):
        pre = pre + denc[k] * w1d[k:k + 1, :]
    rgb_o[...] = _sigmoid(jnp.dot(jnp.maximum(pre, 0.0), rgbw2[...],
                                  preferred_element_type=jnp.float32))


def _run_tc(featx, g3, featr, x, d, embed_a, w1, b1, w2, b2, rgb_w1, rgb_w2,
            norm_w1, norm_w2, sem_w1, sem_w2):
    row = lambda k: pl.BlockSpec((_BN, k), lambda i: (i, 0))
    colmaj = lambda k: pl.BlockSpec((k, _BN), lambda i: (0, i))
    full = lambda a: pl.BlockSpec(a.shape, lambda i: (0,) * len(a.shape))
    w2r = w2.reshape(1, 128)
    w1t = w1.T
    w1x = rgb_w1[0:3]
    w1d = rgb_w1[3:19]
    w1f = rgb_w1[19:83]
    w1a = rgb_w1[83:95]
    b1r = b1.reshape(1, 128)
    b2r = b2.reshape(1, 1)
    args = (featx, g3, featr, x, d, embed_a, w1, b1r, w2, w2r, w1t, w1x, w1d,
            w1f, w1a, rgb_w2, norm_w1, norm_w2, sem_w1, sem_w2, b2r)
    in_specs = [colmaj(featx.shape[0]), colmaj(g3.shape[0]),
                colmaj(featr.shape[0]), row(3), row(3),
                row(embed_a.shape[1])] + [full(a) for a in args[6:]]
    out_shapes = (
        jax.ShapeDtypeStruct((_N, 1), jnp.float32),
        jax.ShapeDtypeStruct((_N, 3), jnp.float32),
        jax.ShapeDtypeStruct((_N, 3), jnp.float32),
        jax.ShapeDtypeStruct((_N, 3), jnp.float32),
        jax.ShapeDtypeStruct((_N, 7), jnp.float32),
        jax.ShapeDtypeStruct((_NBLK, 1, 128), jnp.int32),
    )
    out_specs = (row(1), row(3), row(3), row(3), row(7),
                 pl.BlockSpec((1, 1, 128), lambda i: (i, 0, 0)))
    return pl.pallas_call(
        _tc_body,
        grid=(_NBLK,),
        in_specs=in_specs,
        out_specs=out_specs,
        out_shape=out_shapes,
    )(*args)


def kernel(x, d, embed_a, xyz_table, rgb_table, W1, b1, W2, b2, rgb_W1,
           rgb_W2, norm_W1, norm_W2, sem_W1, sem_W2):
    xx, xy, xz = x[:, 0], x[:, 1], x[:, 2]
    enc_x = _make_grid_encoder(_LX, _LOG2TX, True)
    enc_r = _make_grid_encoder(_LR, _LOG2TR, False)
    xt0 = xyz_table[:, :, 0].reshape(-1)
    xt1 = xyz_table[:, :, 1].reshape(-1)
    rt0 = rgb_table[:, :, 0].reshape(-1)
    rt1 = rgb_table[:, :, 1].reshape(-1)
    featx, g3 = enc_x(xx, xy, xz, xt0, xt1, jnp.asarray(_RES_X))
    (featr,) = enc_r(xx, xy, xz, rt0, rt1, jnp.asarray(_RES_R))
    sig, rgbs, nraw, npred, semv, cntp = _run_tc(
        featx, g3, featr, x, d, embed_a, W1, b1, W2, b2, rgb_W1, rgb_W2,
        norm_W1, norm_W2, sem_W1, sem_W2)
    return (sig[:, 0], rgbs, nraw, npred, semv, jnp.sum(cntp[:, 0, 0]))


# speedup vs baseline: 1.0030x; 1.0030x over previous
import numpy as np
import jax
import jax.numpy as jnp
from jax import lax
from jax.experimental import pallas as pl
from jax.experimental.pallas import tpu as pltpu
from jax.experimental.pallas import tpu_sc as plsc

_N = 65536
_NW = 32
_PW = _N // _NW
_C = 128
_G = _C // 16
_NCHUNK = _PW // _C

_LX, _LOG2TX = 16, 19
_LR, _LOG2TR = 32, 21
_NMIN = 16
_BX = float(np.exp(np.log(2048.0 / _NMIN) / (_LX - 1)))
_BR = float(np.exp(np.log(2048.0 / _NMIN) / (_LR - 1)))
_RES_X = np.tile(np.array([float(int(np.floor(_NMIN * _BX ** l)))
                           for l in range(_LX)], np.float32)[:, None], (1, 16))
_RES_R = np.tile(np.array([float(int(np.floor(_NMIN * _BR ** l)))
                           for l in range(_LR)], np.float32)[:, None], (1, 16))
_P2 = 2654435761 - (1 << 32)
_P3 = 805459861

_CORNERS = [(cx, cy, cz) for cx in (0, 1) for cy in (0, 1) for cz in (0, 1)]


def _make_grid_encoder(n_levels, log2t, want_grad):
    t_sz = 1 << log2t
    mask = t_sz - 1
    no = 2 * n_levels

    outs = [jax.ShapeDtypeStruct((no, _N), jnp.float32)]
    if want_grad:
        outs.append(jax.ShapeDtypeStruct((3 * no, _N), jnp.float32))

    scratch = (
        [pltpu.VMEM((_C,), jnp.float32) for _ in range(3)]
        + [pltpu.VMEM((_C,), jnp.int32) for _ in range(16)]
        + [pltpu.VMEM((_C,), jnp.float32) for _ in range(32)]
        + [pltpu.VMEM((no, _C), jnp.float32)]
        + ([pltpu.VMEM((3 * no, _C), jnp.float32)] if want_grad else [])
        + [pltpu.VMEM((n_levels, 16), jnp.float32),
           pltpu.SemaphoreType.DMA, pltpu.SemaphoreType.DMA]
    )
    mesh = plsc.VectorSubcoreMesh(core_axis_name="c", subcore_axis_name="s")

    def body(*refs):
        xx_h, xy_h, xz_h, tab0_h, tab1_h, res_h = refs[0:6]
        pos = 6
        feat_h = refs[pos]; pos += 1
        if want_grad:
            g3_h = refs[pos]; pos += 1
        xv = refs[pos:pos + 3]; pos += 3
        idxr = refs[pos:pos + 16]; pos += 16
        rows = refs[pos:pos + 32]; pos += 32
        featb = refs[pos]; pos += 1
        if want_grad:
            g3b = refs[pos]; pos += 1
        resv = refs[pos]; pos += 1
        sems = refs[pos:pos + 2]; pos += 2

        wid = lax.axis_index("s") * 2 + lax.axis_index("c")
        base = wid * _PW
        pltpu.sync_copy(res_h, resv)

        def posw(g, resvec):
            out = []
            for ref in xv:
                xn = (ref[pl.ds(g * 16, 16)] + 1.0) * 0.5
                p = xn * resvec
                pi = p.astype(jnp.int32)
                w = p - pi.astype(jnp.float32)
                out.append((pi, w))
            return out

        def chunk_body(k, carry):
            rowstart = base + k * _C
            pltpu.sync_copy(xx_h.at[pl.ds(rowstart, _C)], xv[0])
            pltpu.sync_copy(xy_h.at[pl.ds(rowstart, _C)], xv[1])
            pltpu.sync_copy(xz_h.at[pl.ds(rowstart, _C)], xv[2])

            def fire(l, st):
                resvec = resv[l]
                lofs = l * t_sz

                def genidx(g, c3):
                    (pix, _), (piy, _), (piz, _) = posw(g, resvec)
                    hx = (pix, pix + 1)
                    hy0 = piy * _P2
                    hy = (hy0, hy0 + _P2)
                    hz0 = piz * _P3
                    hz = (hz0, hz0 + _P3)
                    s = pl.ds(g * 16, 16)
                    for c, (cx, cy, cz) in enumerate(_CORNERS):
                        idxr[8 * st + c][s] = ((((hx[cx] ^ hy[cy]) ^ hz[cz])
                                                & mask) + lofs)
                    return c3

                lax.fori_loop(0, _G, genidx, 0)
                for c in range(8):
                    pltpu.async_copy(tab0_h.at[idxr[8 * st + c]],
                                     rows[16 * st + 2 * c], sems[st])
                    pltpu.async_copy(tab1_h.at[idxr[8 * st + c]],
                                     rows[16 * st + 2 * c + 1], sems[st])

            def drain(st):
                for c in range(8):
                    pltpu.make_async_copy(tab0_h.at[idxr[8 * st + c]],
                                          rows[16 * st + 2 * c],
                                          sems[st]).wait()
                    pltpu.make_async_copy(tab1_h.at[idxr[8 * st + c]],
                                          rows[16 * st + 2 * c + 1],
                                          sems[st]).wait()

            def accum_level(l, st):
                resvec = resv[l]

                def accum(g, c3):
                    (pix, wx1), (piy, wy1), (piz, wz1) = posw(g, resvec)
                    wx0, wy0, wz0 = 1.0 - wx1, 1.0 - wy1, 1.0 - wz1
                    s = pl.ds(g * 16, 16)
                    if want_grad:
                        fres = resvec * 0.5
                        for p in (0, 1):
                            f = [rows[16 * st + 2 * c + p][s]
                                 for c in range(8)]
                            m00 = wz0 * f[0] + wz1 * f[1]
                            m01 = wz0 * f[2] + wz1 * f[3]
                            m10 = wz0 * f[4] + wz1 * f[5]
                            m11 = wz0 * f[6] + wz1 * f[7]
                            d00 = f[1] - f[0]
                            d01 = f[3] - f[2]
                            d10 = f[5] - f[4]
                            d11 = f[7] - f[6]
                            n0 = wy0 * m00 + wy1 * m01
                            n1 = wy0 * m10 + wy1 * m11
                            r0 = wy0 * d00 + wy1 * d01
                            r1 = wy0 * d10 + wy1 * d11
                            q0 = m01 - m00
                            q1 = m11 - m10
                            featb[2 * l + p, s] = wx0 * n0 + wx1 * n1
                            g3b[2 * l + p, s] = (n1 - n0) * fres
                            g3b[no + 2 * l + p, s] = (wx0 * q0
                                                      + wx1 * q1) * fres
                            g3b[2 * no + 2 * l + p, s] = (wx0 * r0
                                                          + wx1 * r1) * fres
                        return c3
                    wx = (wx0, wx1)
                    wyz = ((wy0 * wz0, wy0 * wz1), (wy1 * wz0, wy1 * wz1))
                    zf = wx1 * 0.0
                    acc0 = zf
                    acc1 = zf
                    for c, (cx, cy, cz) in enumerate(_CORNERS):
                        f0 = rows[16 * st + 2 * c][s]
                        f1 = rows[16 * st + 2 * c + 1][s]
                        w3 = wx[cx] * wyz[cy][cz]
                        acc0 = acc0 + w3 * f0
                        acc1 = acc1 + w3 * f1
                    featb[2 * l, s] = acc0
                    featb[2 * l + 1, s] = acc1
                    return c3

                lax.fori_loop(0, _G, accum, 0)

            fire(0, 0)

            def pair_body(i, carry2):
                l0 = 2 * i
                fire(l0 + 1, 1)
                drain(0)
                accum_level(l0, 0)

                @pl.when(i + 1 < n_levels // 2)
                def _():
                    fire(l0 + 2, 0)

                drain(1)
                accum_level(l0 + 1, 1)
                return carry2

            lax.fori_loop(0, n_levels // 2, pair_body, 0)
            pltpu.sync_copy(featb, feat_h.at[:, pl.ds(rowstart, _C)])
            if want_grad:
                pltpu.sync_copy(g3b, g3_h.at[:, pl.ds(rowstart, _C)])
            return carry

        lax.fori_loop(0, _NCHUNK, chunk_body, 0)

    return pl.kernel(body, out_type=tuple(outs), mesh=mesh,
                     scratch_types=scratch)


_BN = 2048
_NBLK = _N // _BN


def _softplus(v):
    return jnp.maximum(v, 0.0) + jnp.log(1.0 + jnp.exp(-jnp.abs(v)))


def _sigmoid(v):
    return 1.0 / (1.0 + jnp.exp(-v))


def _tc_body(featx, g3, featr, xb, db, ea, w1, b1, w2, w2r, w1t, w1x, w1d,
             w1f, w1a, rgbw2, nw1, nw2, sw1, sw2, b2,
             sig_o, rgb_o, nraw_o, npred_o, sem_o, cnt_o):
    f = featx[...].T
    pre1 = jnp.dot(f, w1[...], preferred_element_type=jnp.float32) + b1[...]
    h1 = _softplus(pre1)
    h2 = jnp.dot(h1, w2[...], preferred_element_type=jnp.float32) + b2[...]
    sig_o[...] = _softplus(100.0 * h2) / 100.0

    dh1 = _sigmoid(100.0 * h2) * w2r[...]
    dpre1 = dh1 * _sigmoid(pre1)
    g_f = jnp.dot(dpre1, w1t[...], preferred_element_type=jnp.float32)
    g3v = g3[...].T
    no = g_f.shape[1]
    gx = jnp.sum(g3v[:, 0:no] * g_f, axis=1, keepdims=True)
    gy = jnp.sum(g3v[:, no:2 * no] * g_f, axis=1, keepdims=True)
    gz = jnp.sum(g3v[:, 2 * no:3 * no] * g_f, axis=1, keepdims=True)
    gn = jnp.maximum(jnp.sqrt(gx * gx + gy * gy + gz * gz), 1e-6)
    nraw_o[:, 0:1] = -gx / gn
    nraw_o[:, 1:2] = -gy / gn
    nraw_o[:, 2:3] = -gz / gn
    cntv = jnp.sum(jnp.isinf(gx).astype(jnp.int32) +
                   jnp.isinf(gy).astype(jnp.int32) +
                   jnp.isinf(gz).astype(jnp.int32))
    cnt_o[...] = lax.broadcast_in_dim(cntv, (1, 1, 128), ())

    fr = featr[...].T
    hp = jnp.dot(jnp.maximum(jnp.dot(fr, nw1[...],
                                     preferred_element_type=jnp.float32), 0.0),
                 nw2[...], preferred_element_type=jnp.float32)
    hn = jnp.maximum(jnp.sqrt(jnp.sum(hp * hp, axis=1, keepdims=True)), 1e-6)
    npred_o[...] = -hp / hn

    sm = jnp.dot(jnp.maximum(jnp.dot(fr, sw1[...],
                                     preferred_element_type=jnp.float32), 0.0),
                 sw2[...], preferred_element_type=jnp.float32)
    sm = sm - jnp.max(sm, axis=1, keepdims=True)
    es = jnp.exp(sm)
    sem_o[...] = es / jnp.sum(es, axis=1, keepdims=True)

    dv = db[...]
    dn = dv / jnp.maximum(jnp.sqrt(jnp.sum(dv * dv, axis=1, keepdims=True)),
                          1e-6)
    u = (dn + 1.0) / 2.0
    sx = 2.0 * u[:, 0:1] - 1.0
    sy = 2.0 * u[:, 1:2] - 1.0
    sz = 2.0 * u[:, 2:3] - 1.0
    x2, y2, z2 = sx * sx, sy * sy, sz * sz
    xy, yz, xz = sx * sy, sy * sz, sx * sz
    denc = [
        None,
        -0.48860251190291987 * sy,
        0.48860251190291987 * sz,
        -0.48860251190291987 * sx,
        1.0925484305920792 * xy,
        -1.0925484305920792 * yz,
        0.94617469575755997 * z2 - 0.31539156525251999,
        -1.0925484305920792 * xz,
        0.54627421529603959 * (x2 - y2),
        0.59004358992664352 * sy * (-3.0 * x2 + y2),
        2.8906114426405538 * xy * sz,
        0.45704579946446572 * sy * (1.0 - 5.0 * z2),
        0.3731763325901154 * sz * (5.0 * z2 - 3.0),
        0.45704579946446572 * sx * (1.0 - 5.0 * z2),
        1.4453057213202769 * sz * (x2 - y2),
        0.59004358992664352 * sx * (x2 - 3.0 * y2),
    ]
    pre = (jnp.dot(xb[...], w1x[...], preferred_element_type=jnp.float32)
           + jnp.dot(fr, w1f[...], preferred_element_type=jnp.float32)
           + jnp.dot(ea[...], w1a[...], preferred_element_type=jnp.float32))
    pre = pre + 0.28209479177387814 * w1d[0:1, :]
    for k in range(1, 16):
        pre = pre + denc[k] * w1d[k:k + 1, :]
    rgb_o[...] = _sigmoid(jnp.dot(jnp.maximum(pre, 0.0), rgbw2[...],
                                  preferred_element_type=jnp.float32))


def _run_tc(featx, g3, featr, x, d, embed_a, w1, b1, w2, b2, rgb_w1, rgb_w2,
            norm_w1, norm_w2, sem_w1, sem_w2):
    row = lambda k: pl.BlockSpec((_BN, k), lambda i: (i, 0))
    colmaj = lambda k: pl.BlockSpec((k, _BN), lambda i: (0, i))
    full = lambda a: pl.BlockSpec(a.shape, lambda i: (0,) * len(a.shape))
    w2r = w2.reshape(1, 128)
    w1t = w1.T
    w1x = rgb_w1[0:3]
    w1d = rgb_w1[3:19]
    w1f = rgb_w1[19:83]
    w1a = rgb_w1[83:95]
    b1r = b1.reshape(1, 128)
    b2r = b2.reshape(1, 1)
    args = (featx, g3, featr, x, d, embed_a, w1, b1r, w2, w2r, w1t, w1x, w1d,
            w1f, w1a, rgb_w2, norm_w1, norm_w2, sem_w1, sem_w2, b2r)
    in_specs = [colmaj(featx.shape[0]), colmaj(g3.shape[0]),
                colmaj(featr.shape[0]), row(3), row(3),
                row(embed_a.shape[1])] + [full(a) for a in args[6:]]
    out_shapes = (
        jax.ShapeDtypeStruct((_N, 1), jnp.float32),
        jax.ShapeDtypeStruct((_N, 3), jnp.float32),
        jax.ShapeDtypeStruct((_N, 3), jnp.float32),
        jax.ShapeDtypeStruct((_N, 3), jnp.float32),
        jax.ShapeDtypeStruct((_N, 7), jnp.float32),
        jax.ShapeDtypeStruct((_NBLK, 1, 128), jnp.int32),
    )
    out_specs = (row(1), row(3), row(3), row(3), row(7),
                 pl.BlockSpec((1, 1, 128), lambda i: (i, 0, 0)))
    return pl.pallas_call(
        _tc_body,
        grid=(_NBLK,),
        in_specs=in_specs,
        out_specs=out_specs,
        out_shape=out_shapes,
    )(*args)


def kernel(x, d, embed_a, xyz_table, rgb_table, W1, b1, W2, b2, rgb_W1,
           rgb_W2, norm_W1, norm_W2, sem_W1, sem_W2):
    xx, xy, xz = x[:, 0], x[:, 1], x[:, 2]
    enc_x = _make_grid_encoder(_LX, _LOG2TX, True)
    enc_r = _make_grid_encoder(_LR, _LOG2TR, False)
    xt0 = xyz_table[:, :, 0].reshape(-1)
    xt1 = xyz_table[:, :, 1].reshape(-1)
    rt0 = rgb_table[:, :, 0].reshape(-1)
    rt1 = rgb_table[:, :, 1].reshape(-1)
    featx, g3 = enc_x(xx, xy, xz, xt0, xt1, jnp.asarray(_RES_X))
    (featr,) = enc_r(xx, xy, xz, rt0, rt1, jnp.asarray(_RES_R))
    sig, rgbs, nraw, npred, semv, cntp = _run_tc(
        featx, g3, featr, x, d, embed_a, W1, b1, W2, b2, rgb_W1, rgb_W2,
        norm_W1, norm_W2, sem_W1, sem_W2)
    return (sig[:, 0], rgbs, nraw, npred, semv, jnp.sum(cntp[:, 0, 0]))
